# baseline (speedup 1.0000x reference)
import jax
import jax.numpy as jnp
import numpy as np
from jax.experimental import pallas as pl
from jax.experimental.pallas import tpu as pltpu
from jax.sharding import Mesh, PartitionSpec

INPUT_H, INPUT_W = 300, 300
NMS_MAX = 100
CONF_TH = 0.01
IOU_TH = 0.45
NUM_PRED = 100
N_CLASSES = 21
NEG = -1e9
_N = 5000
_NP = 5120
_C = N_CLASSES - 1
_FLATP = 2048
_CK = 128
_NCK = _NP // _CK
_BIGI = 1e9


def _nms_body(sc_in, box_ref, out_ref, sc_s, xs_s, ys_s, xe_s, ye_s, a2_s):
    R = sc_in.shape[0]
    Bl = box_ref.shape[1]
    off0 = box_ref[0]
    off1 = box_ref[1]
    off2 = box_ref[2]
    off3 = box_ref[3]
    anc0 = box_ref[4]
    anc1 = box_ref[5]
    anc2 = box_ref[6]
    anc3 = box_ref[7]
    var0 = box_ref[8]
    var1 = box_ref[9]
    var2 = box_ref[10]
    var3 = box_ref[11]
    cx = off0 * anc2 * var0 + anc0
    cy = off1 * anc3 * var1 + anc1
    w = anc2 * jnp.exp(off2 * var2)
    h = anc3 * jnp.exp(off3 * var3)
    xs = (cx - 0.5 * w) * INPUT_W
    ys = (cy - 0.5 * h) * INPUT_H
    xe = (cx + 0.5 * w) * INPUT_W
    ye = (cy + 0.5 * h) * INPUT_H

    def rep(a):
        return jnp.reshape(jnp.broadcast_to(a[:, None, :], (Bl, _C, _NP)), (R, _NP))

    xs = rep(xs)
    ys = rep(ys)
    xe = rep(xe)
    ye = rep(ye)
    xs_s[...] = xs
    ys_s[...] = ys
    xe_s[...] = xe
    ye_s[...] = ye
    a2_s[...] = jnp.maximum(xe - xs, 0.0) * jnp.maximum(ye - ys, 0.0)

    sc0 = sc_in[...]
    sc_init = jnp.where(sc0 > CONF_TH, sc0, NEG)
    sc_s[...] = sc_init

    cmax0 = sc_init[:, 0:_CK]
    cidx0 = jnp.zeros((R, _CK), jnp.float32)
    px0c = xs[:, 0:_CK]
    py0c = ys[:, 0:_CK]
    px1c = xe[:, 0:_CK]
    py1c = ye[:, 0:_CK]
    for c in range(1, _NCK):
        s = slice(c * _CK, (c + 1) * _CK)
        ch = sc_init[:, s]
        upd = ch > cmax0
        cmax0 = jnp.maximum(cmax0, ch)
        cidx0 = jnp.where(upd, float(c), cidx0)
        px0c = jnp.where(upd, xs[:, s], px0c)
        py0c = jnp.where(upd, ys[:, s], py0c)
        px1c = jnp.where(upd, xe[:, s], px1c)
        py1c = jnp.where(upd, ye[:, s], py1c)

    laneio = jax.lax.broadcasted_iota(jnp.int32, (R, _CK), 1).astype(jnp.float32)

    def bcast(a):
        return jnp.broadcast_to(a, (R, _CK))

    def step(t, carry):
        cmax, cidx, pc0, pc1, pc2, pc3 = carry
        m = jnp.max(cmax, axis=1, keepdims=True)
        mb = bcast(m)
        colidx = jnp.where(cmax == mb, cidx * float(_CK) + laneio, _BIGI)
        idx = jnp.min(colidx, axis=1, keepdims=True)
        ohcol = colidx == bcast(idx)
        bx0 = jnp.sum(jnp.where(ohcol, pc0, 0.0), axis=1, keepdims=True)
        by0 = jnp.sum(jnp.where(ohcol, pc1, 0.0), axis=1, keepdims=True)
        bx1 = jnp.sum(jnp.where(ohcol, pc2, 0.0), axis=1, keepdims=True)
        by1 = jnp.sum(jnp.where(ohcol, pc3, 0.0), axis=1, keepdims=True)
        area1 = jnp.maximum(bx1 - bx0, 0.0) * jnp.maximum(by1 - by0, 0.0)
        valid = m > CONF_TH
        bx0b = bcast(bx0)
        by0b = bcast(by0)
        bx1b = bcast(bx1)
        by1b = bcast(by1)
        a1b = bcast(area1)
        cmax_n = cidx_n = p0n = p1n = p2n = p3n = None
        for c in range(_NCK):
            s = slice(c * _CK, (c + 1) * _CK)
            xsc = xs_s[:, s]
            ysc = ys_s[:, s]
            xec = xe_s[:, s]
            yec = ye_s[:, s]
            iw = jnp.maximum(jnp.minimum(bx1b, xec) - jnp.maximum(bx0b, xsc), 0.0)
            ih = jnp.maximum(jnp.minimum(by1b, yec) - jnp.maximum(by0b, ysc), 0.0)
            inter = iw * ih
            union = a1b + a2_s[:, s] - inter
            iou = inter / jnp.maximum(union, 1e-8)
            sc_new = jnp.where(iou >= IOU_TH, NEG, sc_s[:, s])
            sc_s[:, s] = sc_new
            if c == 0:
                cmax_n = sc_new
                cidx_n = jnp.zeros((R, _CK), jnp.float32)
                p0n = xsc
                p1n = ysc
                p2n = xec
                p3n = yec
            else:
                upd = sc_new > cmax_n
                cmax_n = jnp.maximum(cmax_n, sc_new)
                cidx_n = jnp.where(upd, float(c), cidx_n)
                p0n = jnp.where(upd, xsc, p0n)
                p1n = jnp.where(upd, ysc, p1n)
                p2n = jnp.where(upd, xec, p2n)
                p3n = jnp.where(upd, yec, p3n)
        vrow = valid[:, 0]
        out_ref[0, t, :] = jnp.where(vrow, m[:, 0], 0.0)
        out_ref[1, t, :] = jnp.where(vrow, bx0[:, 0], 0.0)
        out_ref[2, t, :] = jnp.where(vrow, by0[:, 0], 0.0)
        out_ref[3, t, :] = jnp.where(vrow, bx1[:, 0], 0.0)
        out_ref[4, t, :] = jnp.where(vrow, by1[:, 0], 0.0)
        return cmax_n, cidx_n, p0n, p1n, p2n, p3n

    jax.lax.fori_loop(0, NMS_MAX, step, (cmax0, cidx0, px0c, py0c, px1c, py1c))


def _topk_body(sc_ref, f_ref, out_ref):
    Bl = sc_ref.shape[0]
    iota = jax.lax.broadcasted_iota(jnp.int32, (Bl, _FLATP), 1)
    cls = (iota // NMS_MAX + 1).astype(jnp.float32)
    f0 = f_ref[0]
    f1 = f_ref[1]
    f2 = f_ref[2]
    f3 = f_ref[3]

    def pick(onehot, a):
        return jnp.sum(jnp.where(onehot, a, 0.0), axis=1)

    def step(k, sc):
        m = jnp.max(sc, axis=1, keepdims=True)
        idx = jnp.min(jnp.where(sc == m, iota, _FLATP), axis=1, keepdims=True)
        onehot = iota == idx
        out_ref[0, k, :] = pick(onehot, cls)
        out_ref[1, k, :] = m[:, 0]
        out_ref[2, k, :] = pick(onehot, f0)
        out_ref[3, k, :] = pick(onehot, f1)
        out_ref[4, k, :] = pick(onehot, f2)
        out_ref[5, k, :] = pick(onehot, f3)
        return jnp.where(onehot, NEG, sc)

    jax.lax.fori_loop(0, NUM_PRED, step, sc_ref[...])


def _decode_shard(y_shard):
    Bl = y_shard.shape[0]
    R = Bl * _C
    flat_n = _C * NMS_MAX
    yt = jnp.transpose(y_shard, (2, 0, 1))
    yt = jnp.pad(yt, ((0, 0), (0, 0), (0, _NP - _N)))
    sc_in = jnp.reshape(yt[1:N_CLASSES].transpose(1, 0, 2), (R, _NP))
    box_in = yt[N_CLASSES:]

    sel = pl.pallas_call(
        _nms_body,
        out_shape=jax.ShapeDtypeStruct((5, NMS_MAX, R), jnp.float32),
        scratch_shapes=[pltpu.VMEM((R, _NP), jnp.float32)] * 6,
    )(sc_in, box_in)

    flat = jnp.reshape(
        jnp.transpose(jnp.reshape(sel, (5, NMS_MAX, Bl, _C)), (0, 2, 3, 1)),
        (5, Bl, flat_n),
    )
    sc_flat = jnp.pad(flat[0], ((0, 0), (0, _FLATP - flat_n)), constant_values=NEG)
    f_flat = jnp.pad(flat[1:], ((0, 0), (0, 0), (0, _FLATP - flat_n)))

    top = pl.pallas_call(
        _topk_body,
        out_shape=jax.ShapeDtypeStruct((6, NUM_PRED, Bl), jnp.float32),
    )(sc_flat, f_flat)

    return jnp.transpose(top, (2, 1, 0))


@jax.jit
def kernel(y_pred):
    devs = jax.devices()
    nd = 2 if len(devs) >= 2 and y_pred.shape[0] % 2 == 0 else 1
    mesh = Mesh(np.array(devs[:nd]), ("d",))

    def shard_fn(y_full):
        b = y_full.shape[0] // nd
        i = jax.lax.axis_index("d")
        y_loc = jax.lax.dynamic_slice_in_dim(y_full, i * b, b, axis=0)
        return _decode_shard(y_loc)

    return jax.shard_map(
        shard_fn,
        mesh=mesh,
        in_specs=PartitionSpec(),
        out_specs=PartitionSpec("d"),
        check_vma=False,
    )(y_pred)


# speedup vs baseline: 1.0062x; 1.0062x over previous
import jax
import jax.numpy as jnp
import numpy as np
from jax.experimental import pallas as pl
from jax.experimental.pallas import tpu as pltpu
from jax.sharding import Mesh, PartitionSpec

INPUT_H, INPUT_W = 300, 300
NMS_MAX = 100
CONF_TH = 0.01
IOU_TH = 0.45
NUM_PRED = 100
N_CLASSES = 21
NEG = -1e9
_N = 5000
_NP = 5120
_C = N_CLASSES - 1
_FLATP = 2048
_CK = 128
_NCK = _NP // _CK
_BIGI = 1e9


def _nms_body(sc_in, box_ref, out_ref, sc_s, xs_s, ys_s, xe_s, ye_s, a2_s):
    R = sc_in.shape[0]
    Bl = box_ref.shape[1]
    off0 = box_ref[0]
    off1 = box_ref[1]
    off2 = box_ref[2]
    off3 = box_ref[3]
    anc0 = box_ref[4]
    anc1 = box_ref[5]
    anc2 = box_ref[6]
    anc3 = box_ref[7]
    var0 = box_ref[8]
    var1 = box_ref[9]
    var2 = box_ref[10]
    var3 = box_ref[11]
    cx = off0 * anc2 * var0 + anc0
    cy = off1 * anc3 * var1 + anc1
    w = anc2 * jnp.exp(off2 * var2)
    h = anc3 * jnp.exp(off3 * var3)
    xs = (cx - 0.5 * w) * INPUT_W
    ys = (cy - 0.5 * h) * INPUT_H
    xe = (cx + 0.5 * w) * INPUT_W
    ye = (cy + 0.5 * h) * INPUT_H

    def rep(a):
        return jnp.reshape(jnp.broadcast_to(a[:, None, :], (Bl, _C, _NP)), (R, _NP))

    xs = rep(xs)
    ys = rep(ys)
    xe = rep(xe)
    ye = rep(ye)
    xs_s[...] = xs
    ys_s[...] = ys
    xe_s[...] = xe
    ye_s[...] = ye
    a2_s[...] = jnp.maximum(xe - xs, 0.0) * jnp.maximum(ye - ys, 0.0)

    sc0 = sc_in[...]
    sc_init = jnp.where(sc0 > CONF_TH, sc0, NEG)
    sc_s[...] = sc_init

    cmax0 = sc_init[:, 0:_CK]
    cidx0 = jnp.zeros((R, _CK), jnp.float32)
    px0c = xs[:, 0:_CK]
    py0c = ys[:, 0:_CK]
    px1c = xe[:, 0:_CK]
    py1c = ye[:, 0:_CK]
    for c in range(1, _NCK):
        s = slice(c * _CK, (c + 1) * _CK)
        ch = sc_init[:, s]
        upd = ch > cmax0
        cmax0 = jnp.maximum(cmax0, ch)
        cidx0 = jnp.where(upd, float(c), cidx0)
        px0c = jnp.where(upd, xs[:, s], px0c)
        py0c = jnp.where(upd, ys[:, s], py0c)
        px1c = jnp.where(upd, xe[:, s], px1c)
        py1c = jnp.where(upd, ye[:, s], py1c)

    laneio = jax.lax.broadcasted_iota(jnp.int32, (R, _CK), 1).astype(jnp.float32)

    def bcast(a):
        return jnp.broadcast_to(a, (R, _CK))

    def step(t, carry):
        cmax, cidx, pc0, pc1, pc2, pc3 = carry
        m = jnp.max(cmax, axis=1, keepdims=True)
        mb = bcast(m)
        colidx = jnp.where(cmax == mb, cidx * float(_CK) + laneio, _BIGI)
        idx = jnp.min(colidx, axis=1, keepdims=True)
        lstar = (idx - jnp.floor(idx * (1.0 / _CK)) * float(_CK)).astype(jnp.int32)
        bx0 = jnp.take_along_axis(pc0, lstar, axis=1)
        by0 = jnp.take_along_axis(pc1, lstar, axis=1)
        bx1 = jnp.take_along_axis(pc2, lstar, axis=1)
        by1 = jnp.take_along_axis(pc3, lstar, axis=1)
        area1 = jnp.maximum(bx1 - bx0, 0.0) * jnp.maximum(by1 - by0, 0.0)
        valid = m > CONF_TH
        bx0b = bcast(bx0)
        by0b = bcast(by0)
        bx1b = bcast(bx1)
        by1b = bcast(by1)
        a1b = bcast(area1)
        cmax_n = cidx_n = p0n = p1n = p2n = p3n = None
        for c in range(_NCK):
            s = slice(c * _CK, (c + 1) * _CK)
            xsc = xs_s[:, s]
            ysc = ys_s[:, s]
            xec = xe_s[:, s]
            yec = ye_s[:, s]
            iw = jnp.maximum(jnp.minimum(bx1b, xec) - jnp.maximum(bx0b, xsc), 0.0)
            ih = jnp.maximum(jnp.minimum(by1b, yec) - jnp.maximum(by0b, ysc), 0.0)
            inter = iw * ih
            union = a1b + a2_s[:, s] - inter
            iou = inter / jnp.maximum(union, 1e-8)
            sc_new = jnp.where(iou >= IOU_TH, NEG, sc_s[:, s])
            sc_s[:, s] = sc_new
            if c == 0:
                cmax_n = sc_new
                cidx_n = jnp.zeros((R, _CK), jnp.float32)
                p0n = xsc
                p1n = ysc
                p2n = xec
                p3n = yec
            else:
                upd = sc_new > cmax_n
                cmax_n = jnp.maximum(cmax_n, sc_new)
                cidx_n = jnp.where(upd, float(c), cidx_n)
                p0n = jnp.where(upd, xsc, p0n)
                p1n = jnp.where(upd, ysc, p1n)
                p2n = jnp.where(upd, xec, p2n)
                p3n = jnp.where(upd, yec, p3n)
        vrow = valid[:, 0]
        out_ref[0, t, :] = jnp.where(vrow, m[:, 0], 0.0)
        out_ref[1, t, :] = jnp.where(vrow, bx0[:, 0], 0.0)
        out_ref[2, t, :] = jnp.where(vrow, by0[:, 0], 0.0)
        out_ref[3, t, :] = jnp.where(vrow, bx1[:, 0], 0.0)
        out_ref[4, t, :] = jnp.where(vrow, by1[:, 0], 0.0)
        return cmax_n, cidx_n, p0n, p1n, p2n, p3n

    jax.lax.fori_loop(0, NMS_MAX, step, (cmax0, cidx0, px0c, py0c, px1c, py1c))


def _topk_body(sc_ref, f_ref, out_ref):
    Bl = sc_ref.shape[0]
    iota = jax.lax.broadcasted_iota(jnp.int32, (Bl, _FLATP), 1)
    cls = (iota // NMS_MAX + 1).astype(jnp.float32)
    f0 = f_ref[0]
    f1 = f_ref[1]
    f2 = f_ref[2]
    f3 = f_ref[3]

    def pick(onehot, a):
        return jnp.sum(jnp.where(onehot, a, 0.0), axis=1)

    def step(k, sc):
        m = jnp.max(sc, axis=1, keepdims=True)
        idx = jnp.min(jnp.where(sc == m, iota, _FLATP), axis=1, keepdims=True)
        onehot = iota == idx
        out_ref[0, k, :] = pick(onehot, cls)
        out_ref[1, k, :] = m[:, 0]
        out_ref[2, k, :] = pick(onehot, f0)
        out_ref[3, k, :] = pick(onehot, f1)
        out_ref[4, k, :] = pick(onehot, f2)
        out_ref[5, k, :] = pick(onehot, f3)
        return jnp.where(onehot, NEG, sc)

    jax.lax.fori_loop(0, NUM_PRED, step, sc_ref[...])


def _decode_shard(y_shard):
    Bl = y_shard.shape[0]
    R = Bl * _C
    flat_n = _C * NMS_MAX
    yt = jnp.transpose(y_shard, (2, 0, 1))
    yt = jnp.pad(yt, ((0, 0), (0, 0), (0, _NP - _N)))
    sc_in = jnp.reshape(yt[1:N_CLASSES].transpose(1, 0, 2), (R, _NP))
    box_in = yt[N_CLASSES:]

    sel = pl.pallas_call(
        _nms_body,
        out_shape=jax.ShapeDtypeStruct((5, NMS_MAX, R), jnp.float32),
        scratch_shapes=[pltpu.VMEM((R, _NP), jnp.float32)] * 6,
    )(sc_in, box_in)

    flat = jnp.reshape(
        jnp.transpose(jnp.reshape(sel, (5, NMS_MAX, Bl, _C)), (0, 2, 3, 1)),
        (5, Bl, flat_n),
    )
    sc_flat = jnp.pad(flat[0], ((0, 0), (0, _FLATP - flat_n)), constant_values=NEG)
    f_flat = jnp.pad(flat[1:], ((0, 0), (0, 0), (0, _FLATP - flat_n)))

    top = pl.pallas_call(
        _topk_body,
        out_shape=jax.ShapeDtypeStruct((6, NUM_PRED, Bl), jnp.float32),
    )(sc_flat, f_flat)

    return jnp.transpose(top, (2, 1, 0))


@jax.jit
def kernel(y_pred):
    devs = jax.devices()
    nd = 2 if len(devs) >= 2 and y_pred.shape[0] % 2 == 0 else 1
    mesh = Mesh(np.array(devs[:nd]), ("d",))

    def shard_fn(y_full):
        b = y_full.shape[0] // nd
        i = jax.lax.axis_index("d")
        y_loc = jax.lax.dynamic_slice_in_dim(y_full, i * b, b, axis=0)
        return _decode_shard(y_loc)

    return jax.shard_map(
        shard_fn,
        mesh=mesh,
        in_specs=PartitionSpec(),
        out_specs=PartitionSpec("d"),
        check_vma=False,
    )(y_pred)


# speedup vs baseline: 1.0631x; 1.0566x over previous
import jax
import jax.numpy as jnp
import numpy as np
from jax.experimental import pallas as pl
from jax.experimental.pallas import tpu as pltpu
from jax.sharding import Mesh, PartitionSpec

INPUT_H, INPUT_W = 300, 300
NMS_MAX = 100
CONF_TH = 0.01
IOU_TH = 0.45
NUM_PRED = 100
N_CLASSES = 21
NEG = -1e9
_N = 5000
_NP = 5120
_C = N_CLASSES - 1
_FLATP = 2048
_CK = 128
_NCK = _NP // _CK
_BIGI = 1e9


def _nms_body(sc_in, box_ref, out_ref, sc_s, xs_s, ys_s, xe_s, ye_s, a2_s):
    R = sc_in.shape[0]
    Bl = box_ref.shape[1]
    off0 = box_ref[0]
    off1 = box_ref[1]
    off2 = box_ref[2]
    off3 = box_ref[3]
    anc0 = box_ref[4]
    anc1 = box_ref[5]
    anc2 = box_ref[6]
    anc3 = box_ref[7]
    var0 = box_ref[8]
    var1 = box_ref[9]
    var2 = box_ref[10]
    var3 = box_ref[11]
    cx = off0 * anc2 * var0 + anc0
    cy = off1 * anc3 * var1 + anc1
    w = anc2 * jnp.exp(off2 * var2)
    h = anc3 * jnp.exp(off3 * var3)
    xs = (cx - 0.5 * w) * INPUT_W
    ys = (cy - 0.5 * h) * INPUT_H
    xe = (cx + 0.5 * w) * INPUT_W
    ye = (cy + 0.5 * h) * INPUT_H

    def rep(a):
        return jnp.reshape(jnp.broadcast_to(a[:, None, :], (Bl, _C, _NP)), (R, _NP))

    xs = rep(xs)
    ys = rep(ys)
    xe = rep(xe)
    ye = rep(ye)
    xs_s[...] = xs
    ys_s[...] = ys
    xe_s[...] = xe
    ye_s[...] = ye
    a2_s[...] = jnp.maximum(xe - xs, 0.0) * jnp.maximum(ye - ys, 0.0)

    sc0 = sc_in[...]
    sc_init = jnp.where(sc0 > CONF_TH, sc0, NEG)
    sc_s[...] = sc_init

    cmax0 = sc_init[:, 0:_CK]
    cidx0 = jnp.zeros((R, _CK), jnp.float32)
    px0c = xs[:, 0:_CK]
    py0c = ys[:, 0:_CK]
    px1c = xe[:, 0:_CK]
    py1c = ye[:, 0:_CK]
    for c in range(1, _NCK):
        s = slice(c * _CK, (c + 1) * _CK)
        ch = sc_init[:, s]
        upd = ch > cmax0
        cmax0 = jnp.maximum(cmax0, ch)
        cidx0 = jnp.where(upd, float(c), cidx0)
        px0c = jnp.where(upd, xs[:, s], px0c)
        py0c = jnp.where(upd, ys[:, s], py0c)
        px1c = jnp.where(upd, xe[:, s], px1c)
        py1c = jnp.where(upd, ye[:, s], py1c)

    laneio = jax.lax.broadcasted_iota(jnp.int32, (R, _CK), 1).astype(jnp.float32)

    def bcast(a):
        return jnp.broadcast_to(a, (R, _CK))

    def step(t, carry):
        cmax, cidx, pc0, pc1, pc2, pc3 = carry
        m = jnp.max(cmax, axis=1, keepdims=True)
        mb = bcast(m)
        colidx = jnp.where(cmax == mb, cidx * float(_CK) + laneio, _BIGI)
        idx = jnp.min(colidx, axis=1, keepdims=True)
        lstar = (idx - jnp.floor(idx * (1.0 / _CK)) * float(_CK)).astype(jnp.int32)
        bx0 = jnp.take_along_axis(pc0, lstar, axis=1)
        by0 = jnp.take_along_axis(pc1, lstar, axis=1)
        bx1 = jnp.take_along_axis(pc2, lstar, axis=1)
        by1 = jnp.take_along_axis(pc3, lstar, axis=1)
        area1 = jnp.maximum(bx1 - bx0, 0.0) * jnp.maximum(by1 - by0, 0.0)
        valid = m > CONF_TH
        bx0b = bcast(bx0)
        by0b = bcast(by0)
        bx1b = bcast(bx1)
        by1b = bcast(by1)
        a1b = bcast(area1)
        cmax_n = cidx_n = p0n = p1n = p2n = p3n = None
        for c in range(_NCK):
            s = slice(c * _CK, (c + 1) * _CK)
            xsc = xs_s[:, s]
            ysc = ys_s[:, s]
            xec = xe_s[:, s]
            yec = ye_s[:, s]
            iw = jnp.maximum(jnp.minimum(bx1b, xec) - jnp.maximum(bx0b, xsc), 0.0)
            ih = jnp.maximum(jnp.minimum(by1b, yec) - jnp.maximum(by0b, ysc), 0.0)
            inter = iw * ih
            union = a1b + a2_s[:, s] - inter
            iou = inter / jnp.maximum(union, 1e-8)
            sc_new = jnp.where(iou >= IOU_TH, NEG, sc_s[:, s])
            sc_s[:, s] = sc_new
            if c == 0:
                cmax_n = sc_new
                cidx_n = jnp.zeros((R, _CK), jnp.float32)
                p0n = xsc
                p1n = ysc
                p2n = xec
                p3n = yec
            else:
                upd = sc_new > cmax_n
                cmax_n = jnp.maximum(cmax_n, sc_new)
                cidx_n = jnp.where(upd, float(c), cidx_n)
                p0n = jnp.where(upd, xsc, p0n)
                p1n = jnp.where(upd, ysc, p1n)
                p2n = jnp.where(upd, xec, p2n)
                p3n = jnp.where(upd, yec, p3n)
        vrow = valid[:, 0]
        out_ref[0, t, :] = jnp.where(vrow, m[:, 0], 0.0)
        out_ref[1, t, :] = jnp.where(vrow, bx0[:, 0], 0.0)
        out_ref[2, t, :] = jnp.where(vrow, by0[:, 0], 0.0)
        out_ref[3, t, :] = jnp.where(vrow, bx1[:, 0], 0.0)
        out_ref[4, t, :] = jnp.where(vrow, by1[:, 0], 0.0)
        return cmax_n, cidx_n, p0n, p1n, p2n, p3n

    jax.lax.fori_loop(0, NMS_MAX, step, (cmax0, cidx0, px0c, py0c, px1c, py1c))


def _topk_body(sc_ref, f_ref, out_ref):
    RR = sc_ref.shape[0]
    Bl = RR // 16
    laneio_i = jax.lax.broadcasted_iota(jnp.int32, (RR, _CK), 1)
    laneio = laneio_i.astype(jnp.float32)
    rowf = (
        (jax.lax.broadcasted_iota(jnp.int32, (RR, 1), 0) % 16) * _CK
    ).astype(jnp.float32)
    f0 = f_ref[0]
    f1 = f_ref[1]
    f2 = f_ref[2]
    f3 = f_ref[3]

    def seg_reduce(a, fn):
        r = fn(jnp.reshape(a, (Bl, 16, 1)), axis=1, keepdims=True)
        return jnp.reshape(jnp.broadcast_to(r, (Bl, 16, 1)), (RR, 1))

    def step(k, carry):
        sc, o0, o1, o2, o3, o4, o5 = carry
        mrow = jnp.max(sc, axis=1, keepdims=True)
        mseg = seg_reduce(mrow, jnp.max)
        candf = jnp.where(
            sc == jnp.broadcast_to(mseg, (RR, _CK)), rowf + laneio, _BIGI
        )
        fmin = jnp.min(candf, axis=1, keepdims=True)
        fwin = seg_reduce(fmin, jnp.min)
        fbase = jnp.floor(fwin * (1.0 / _CK)) * float(_CK)
        lstar = (fwin - fbase).astype(jnp.int32)
        rowsel = rowf == fbase
        g0 = jnp.take_along_axis(f0, lstar, axis=1)
        g1 = jnp.take_along_axis(f1, lstar, axis=1)
        g2 = jnp.take_along_axis(f2, lstar, axis=1)
        g3 = jnp.take_along_axis(f3, lstar, axis=1)
        v0 = seg_reduce(jnp.where(rowsel, g0, NEG), jnp.max)
        v1 = seg_reduce(jnp.where(rowsel, g1, NEG), jnp.max)
        v2 = seg_reduce(jnp.where(rowsel, g2, NEG), jnp.max)
        v3 = seg_reduce(jnp.where(rowsel, g3, NEG), jnp.max)
        cls = jnp.floor((fwin + 0.5) * (1.0 / NMS_MAX)) + 1.0
        lm = laneio_i == k
        o0 = jnp.where(lm, jnp.broadcast_to(cls, (RR, _CK)), o0)
        o1 = jnp.where(lm, jnp.broadcast_to(mseg, (RR, _CK)), o1)
        o2 = jnp.where(lm, jnp.broadcast_to(v0, (RR, _CK)), o2)
        o3 = jnp.where(lm, jnp.broadcast_to(v1, (RR, _CK)), o3)
        o4 = jnp.where(lm, jnp.broadcast_to(v2, (RR, _CK)), o4)
        o5 = jnp.where(lm, jnp.broadcast_to(v3, (RR, _CK)), o5)
        kill = rowsel & (laneio == jnp.broadcast_to(fwin - fbase, (RR, _CK)))
        sc = jnp.where(kill, NEG, sc)
        return sc, o0, o1, o2, o3, o4, o5

    z = jnp.zeros((RR, _CK), jnp.float32)
    _, o0, o1, o2, o3, o4, o5 = jax.lax.fori_loop(
        0, NUM_PRED, step, (sc_ref[...], z, z, z, z, z, z)
    )
    out_ref[0] = o0
    out_ref[1] = o1
    out_ref[2] = o2
    out_ref[3] = o3
    out_ref[4] = o4
    out_ref[5] = o5


def _decode_shard(y_shard):
    Bl = y_shard.shape[0]
    R = Bl * _C
    flat_n = _C * NMS_MAX
    yt = jnp.transpose(y_shard, (2, 0, 1))
    yt = jnp.pad(yt, ((0, 0), (0, 0), (0, _NP - _N)))
    sc_in = jnp.reshape(yt[1:N_CLASSES].transpose(1, 0, 2), (R, _NP))
    box_in = yt[N_CLASSES:]

    sel = pl.pallas_call(
        _nms_body,
        out_shape=jax.ShapeDtypeStruct((5, NMS_MAX, R), jnp.float32),
        scratch_shapes=[pltpu.VMEM((R, _NP), jnp.float32)] * 6,
    )(sc_in, box_in)

    flat = jnp.reshape(
        jnp.transpose(jnp.reshape(sel, (5, NMS_MAX, Bl, _C)), (0, 2, 3, 1)),
        (5, Bl, flat_n),
    )
    sc_flat = jnp.pad(flat[0], ((0, 0), (0, _FLATP - flat_n)), constant_values=NEG)
    f_flat = jnp.pad(flat[1:], ((0, 0), (0, 0), (0, _FLATP - flat_n)))
    sc_rows = jnp.reshape(sc_flat, (Bl * 16, _CK))
    f_rows = jnp.reshape(f_flat, (4, Bl * 16, _CK))

    top = pl.pallas_call(
        _topk_body,
        out_shape=jax.ShapeDtypeStruct((6, Bl * 16, _CK), jnp.float32),
    )(sc_rows, f_rows)

    return jnp.transpose(top[:, ::16, :NUM_PRED], (1, 2, 0))


@jax.jit
def kernel(y_pred):
    devs = jax.devices()
    nd = 2 if len(devs) >= 2 and y_pred.shape[0] % 2 == 0 else 1
    mesh = Mesh(np.array(devs[:nd]), ("d",))

    def shard_fn(y_full):
        b = y_full.shape[0] // nd
        i = jax.lax.axis_index("d")
        y_loc = jax.lax.dynamic_slice_in_dim(y_full, i * b, b, axis=0)
        return _decode_shard(y_loc)

    return jax.shard_map(
        shard_fn,
        mesh=mesh,
        in_specs=PartitionSpec(),
        out_specs=PartitionSpec("d"),
        check_vma=False,
    )(y_pred)
